# baseline (speedup 1.0000x reference)
import jax
import jax.numpy as jnp
from jax.experimental import pallas as pl


def _matvec_body(d_ref, w_ref, b_ref, o_ref):
    d = d_ref[0]
    w = w_ref[0]
    r = jax.lax.dot_general(
        w, d,
        dimension_numbers=(((1,), (1,)), ((), ())),
        preferred_element_type=jnp.float32,
    )
    o_ref[0] = r + b_ref[0, 0, 0]


def kernel(dict, coef):
    D = dict
    B, L, K = D.shape
    w = coef[:, :K].reshape(B, 1, K)
    bias = jnp.broadcast_to(coef[:, K:].reshape(B, 1, 1), (B, 1, 128))

    out = pl.pallas_call(
        _matvec_body,
        grid=(B,),
        in_specs=[
            pl.BlockSpec((1, L, K), lambda b: (b, 0, 0)),
            pl.BlockSpec((1, 1, K), lambda b: (b, 0, 0)),
            pl.BlockSpec((1, 1, 128), lambda b: (b, 0, 0)),
        ],
        out_specs=pl.BlockSpec((1, 1, L), lambda b: (b, 0, 0)),
        out_shape=jax.ShapeDtypeStruct((B, 1, L), jnp.float32),
    )(D, w, bias)
    return out.reshape(B, L, 1)
